# Initial kernel scaffold; baseline (speedup 1.0000x reference)
import jax
import jax.numpy as jnp
from jax.experimental import pallas as pl


def kernel(x, params):
    raise NotImplementedError("write your pallas kernel here")


# speedup vs baseline: 1.7650x; 1.7650x over previous
import functools

import jax
import jax.numpy as jnp
from jax.experimental import pallas as pl

F32 = jnp.float32
HI = jax.lax.Precision.HIGHEST

_B, _N = 2, 1024
_PLANES = [32, 64, 128, 256, 512]
_NSAMPLE = [8, 16, 16, 16, 16]
_EPS = 1e-5
_MT = [128, 64, 64, 64, 64]
_MA = [128, 128, 128, 64, 32]


def _dot(a, b):
    return jax.lax.dot_general(a, b, (((1,), (0,)), ((), ())),
                               precision=HI, preferred_element_type=F32)


def _knn_body(xm_ref, xat_ref, idx_ref, *, mk, n, k):
    xm = xm_ref[0]
    xat = xat_ref[0]
    d = -2.0 * _dot(xm, xat)
    d = d + jnp.sum(xm * xm, axis=1, keepdims=True)
    d = d + jnp.sum(xat * xat, axis=0, keepdims=True)
    lane = jax.lax.broadcasted_iota(jnp.int32, (mk, n), 1)
    kcol = jax.lax.broadcasted_iota(jnp.int32, (mk, k), 1)
    idx_acc = jnp.zeros((mk, k), jnp.int32)
    for j in range(k):
        dmin = jnp.min(d, axis=1, keepdims=True)
        amin = jnp.min(jnp.where(d == dmin, lane, n), axis=1, keepdims=True)
        idx_acc = jnp.where(kcol == j, amin, idx_acc)
        d = jnp.where(lane == amin, jnp.float32(jnp.inf), d)
    idx_ref[0] = idx_acc


def _knn(xyz, xyz_t, mk=256):
    return pl.pallas_call(
        functools.partial(_knn_body, mk=mk, n=_N, k=16),
        grid=(_B, _N // mk),
        in_specs=[
            pl.BlockSpec((1, mk, 3), lambda b, i: (b, i, 0)),
            pl.BlockSpec((1, 3, _N), lambda b, i: (b, 0, 0)),
        ],
        out_specs=pl.BlockSpec((1, mk, 16), lambda b, i: (b, i, 0)),
        out_shape=jax.ShapeDtypeStruct((_B, _N, 16), jnp.int32),
    )(xyz, xyz_t)


def _td_body(ftab_ref, idx_ref, w_ref, hmax_ref, hmin_ref, stats_ref,
             *, mt, n, k, cout):
    step = pl.program_id(0) * pl.num_programs(1) + pl.program_id(1)
    r = mt * k
    ftab = ftab_ref[0]
    idxv = idx_ref[0]
    lane = jax.lax.broadcasted_iota(jnp.int32, (r, n), 1)
    onehot = (idxv == lane).astype(F32)
    g = _dot(onehot, ftab)
    h = _dot(g, w_ref[...])
    h3 = h.reshape(mt, k, cout)
    hmax_ref[0] = jnp.max(h3, axis=1)
    hmin_ref[0] = jnp.min(h3, axis=1)

    @pl.when(step == 0)
    def _():
        stats_ref[...] = jnp.zeros_like(stats_ref)

    stats_ref[0:1, :] += jnp.sum(h, axis=0, keepdims=True)
    stats_ref[1:2, :] += jnp.sum(h * h, axis=0, keepdims=True)


def _td(feature, idx_flat, w, *, k, mt):
    cin = feature.shape[-1]
    cout = w.shape[-1]
    return pl.pallas_call(
        functools.partial(_td_body, mt=mt, n=_N, k=k, cout=cout),
        grid=(_B, _N // mt),
        in_specs=[
            pl.BlockSpec((1, _N, cin), lambda b, i: (b, 0, 0)),
            pl.BlockSpec((1, mt * k, 1), lambda b, i: (b, i, 0)),
            pl.BlockSpec((cin, cout), lambda b, i: (0, 0)),
        ],
        out_specs=[
            pl.BlockSpec((1, mt, cout), lambda b, i: (b, i, 0)),
            pl.BlockSpec((1, mt, cout), lambda b, i: (b, i, 0)),
            pl.BlockSpec((8, cout), lambda b, i: (0, 0)),
        ],
        out_shape=[
            jax.ShapeDtypeStruct((_B, _N, cout), F32),
            jax.ShapeDtypeStruct((_B, _N, cout), F32),
            jax.ShapeDtypeStruct((8, cout), F32),
        ],
    )(feature, idx_flat, w)


def _proj_body(hmax_ref, hmin_ref, stats_ref, gamma_ref, beta_ref,
               wq_ref, wk_ref, wv_ref, f_ref, q_ref, kf_ref, vf_ref, *, cnt):
    s1 = stats_ref[0:1, :]
    s2 = stats_ref[1:2, :]
    m = s1 / cnt
    v = s2 / cnt - m * m
    a = gamma_ref[...] / jnp.sqrt(v + _EPS)
    c = beta_ref[...] - a * m
    hmax = hmax_ref[0]
    hmin = hmin_ref[0]
    f = jnp.maximum(jnp.maximum(a * hmax + c, a * hmin + c), 0.0)
    f_ref[0] = f
    q_ref[0] = _dot(f, wq_ref[...])
    kf_ref[0] = _dot(f, wk_ref[...])
    vf_ref[0] = _dot(f, wv_ref[...])


def _proj(hmax, hmin, stats, gamma, beta, wq, wk, wv, *, cnt, mp=128):
    d = wq.shape[0]
    spec_t = pl.BlockSpec((1, mp, d), lambda b, i: (b, i, 0))
    spec_w = pl.BlockSpec((d, d), lambda b, i: (0, 0))
    spec_v = pl.BlockSpec((1, d), lambda b, i: (0, 0))
    out_sds = jax.ShapeDtypeStruct((_B, _N, d), F32)
    return pl.pallas_call(
        functools.partial(_proj_body, cnt=float(cnt)),
        grid=(_B, _N // mp),
        in_specs=[
            spec_t, spec_t,
            pl.BlockSpec((8, d), lambda b, i: (0, 0)),
            spec_v, spec_v, spec_w, spec_w, spec_w,
        ],
        out_specs=[spec_t, spec_t, spec_t, spec_t],
        out_shape=[out_sds, out_sds, out_sds, out_sds],
    )(hmax, hmin, stats, gamma.reshape(1, d), beta.reshape(1, d), wq, wk, wv)


def _attn_body(idx_ref, q_ref, f_ref, xm_ref, kf_tab_ref, vf_tab_ref,
               xyz_tab_ref, wp1_ref, bp1_ref, wp2_ref, bp2_ref, wa1_ref,
               ba1_ref, wa2_ref, ba2_ref, wf_ref, bf_ref, out_ref,
               *, m, n, k, d):
    r = m * k
    idxv = idx_ref[0]
    lane = jax.lax.broadcasted_iota(jnp.int32, (r, n), 1)
    onehot = (idxv == lane).astype(F32)
    kf = _dot(onehot, kf_tab_ref[0])
    vf = _dot(onehot, vf_tab_ref[0])
    nxyz = _dot(onehot, xyz_tab_ref[0])
    xm = xm_ref[0]
    xm_exp = jnp.broadcast_to(xm[:, None, :], (m, k, 3)).reshape(r, 3)
    rel = xm_exp - nxyz
    ph = jnp.maximum(_dot(rel, wp1_ref[...]) + bp1_ref[...], 0.0)
    pos = _dot(ph, wp2_ref[...]) + bp2_ref[...]
    q = q_ref[0]
    q_exp = jnp.broadcast_to(q[:, None, :], (m, k, d)).reshape(r, d)
    u = q_exp - kf + pos
    hid = jnp.maximum(_dot(u, wa1_ref[...]) + ba1_ref[...], 0.0)
    attn = _dot(hid, wa2_ref[...]) + ba2_ref[...]
    a3 = (attn / jnp.sqrt(jnp.float32(d))).reshape(m, k, d)
    a3 = a3 - jax.lax.stop_gradient(jnp.max(a3, axis=1, keepdims=True))
    e = jnp.exp(a3)
    sm = e / jnp.sum(e, axis=1, keepdims=True)
    v3 = (vf + pos).reshape(m, k, d)
    nf = jnp.sum(sm * v3, axis=1)
    out_ref[0] = _dot(nf, wf_ref[...]) + bf_ref[...] + f_ref[0]


def _attn(idx_flat, q, f, xyz, kf, vf, bp, *, k, m):
    d = q.shape[-1]
    spec_t = pl.BlockSpec((1, m, d), lambda b, i: (b, i, 0))
    spec_v = lambda dim: pl.BlockSpec((1, dim), lambda b, i: (0, 0))
    spec_w = lambda din, dout: pl.BlockSpec((din, dout), lambda b, i: (0, 0))
    return pl.pallas_call(
        functools.partial(_attn_body, m=m, n=_N, k=k, d=d),
        grid=(_B, _N // m),
        in_specs=[
            pl.BlockSpec((1, m * k, 1), lambda b, i: (b, i, 0)),
            spec_t,
            spec_t,
            pl.BlockSpec((1, m, 3), lambda b, i: (b, i, 0)),
            pl.BlockSpec((1, _N, d), lambda b, i: (b, 0, 0)),
            pl.BlockSpec((1, _N, d), lambda b, i: (b, 0, 0)),
            pl.BlockSpec((1, _N, 3), lambda b, i: (b, 0, 0)),
            spec_w(3, 64), spec_v(64),
            spec_w(64, d), spec_v(d),
            spec_w(d, 4 * d), spec_v(4 * d),
            spec_w(4 * d, d), spec_v(d),
            spec_w(d, d), spec_v(d),
        ],
        out_specs=spec_t,
        out_shape=jax.ShapeDtypeStruct((_B, _N, d), F32),
    )(idx_flat, q, f, xyz, kf, vf, xyz,
      bp["Wp1"], bp["bp1"].reshape(1, 64),
      bp["Wp2"], bp["bp2"].reshape(1, d),
      bp["Wa1"], bp["ba1"].reshape(1, 4 * d),
      bp["Wa2"], bp["ba2"].reshape(1, d),
      bp["Wf"], bp["bf"].reshape(1, d))


def _head_body(f_ref, w1_ref, b1_ref, g1_ref, be1_ref, w2_ref, b2_ref,
               g2_ref, be2_ref, w3_ref, b3_ref, out_ref):
    rows = [jnp.mean(f_ref[b], axis=0, keepdims=True) for b in range(_B)]
    h = jnp.concatenate(rows, axis=0)

    def bn(x, g, be):
        mu = jnp.mean(x, axis=0, keepdims=True)
        va = jnp.mean((x - mu) * (x - mu), axis=0, keepdims=True)
        return g * (x - mu) / jnp.sqrt(va + _EPS) + be

    h = jnp.maximum(bn(_dot(h, w1_ref[...]) + b1_ref[...],
                       g1_ref[...], be1_ref[...]), 0.0)
    h = jnp.maximum(bn(_dot(h, w2_ref[...]) + b2_ref[...],
                       g2_ref[...], be2_ref[...]), 0.0)
    out_ref[...] = _dot(h, w3_ref[...]) + b3_ref[...]


def _head(feature, c):
    spec_w = lambda din, dout: pl.BlockSpec((din, dout), lambda: (0, 0))
    spec_v = lambda dim: pl.BlockSpec((1, dim), lambda: (0, 0))
    return pl.pallas_call(
        _head_body,
        in_specs=[
            pl.BlockSpec((_B, _N, 512), lambda: (0, 0, 0)),
            spec_w(512, 256), spec_v(256), spec_v(256), spec_v(256),
            spec_w(256, 128), spec_v(128), spec_v(128), spec_v(128),
            spec_w(128, 40), spec_v(40),
        ],
        out_specs=pl.BlockSpec((_B, 40), lambda: (0, 0)),
        out_shape=jax.ShapeDtypeStruct((_B, 40), F32),
    )(feature,
      c["W1"], c["b1"].reshape(1, 256), c["g1"].reshape(1, 256),
      c["be1"].reshape(1, 256),
      c["W2"], c["b2"].reshape(1, 128), c["g2"].reshape(1, 128),
      c["be2"].reshape(1, 128),
      c["W3"], c["b3"].reshape(1, 40))


def kernel(x, params):
    xyz = x[..., :3]
    xyz_t = jnp.swapaxes(xyz, 1, 2)
    idx16 = _knn(xyz, xyz_t)
    idx_flat = {
        16: idx16.reshape(_B, _N * 16, 1),
        8: idx16[:, :, :8].reshape(_B, _N * 8, 1),
    }
    feature = x
    for i, st in enumerate(params["stages"]):
        k = _NSAMPLE[i]
        td = st["td"]
        hmax, hmin, stats = _td(feature, idx_flat[k], td["W"], k=k, mt=_MT[i])
        bp = st["blocks"][0]
        f, q, kf, vf = _proj(hmax, hmin, stats, td["gamma"], td["beta"],
                             bp["Wq"], bp["Wk"], bp["Wv"], cnt=_B * _N * k)
        feature = _attn(idx_flat[k], q, f, xyz, kf, vf, bp, k=k, m=_MA[i])
    return _head(feature, params["cls"])


# speedup vs baseline: 8.3268x; 4.7177x over previous
import functools

import jax
import jax.numpy as jnp
from jax.experimental import pallas as pl

F32 = jnp.float32
HI = jax.lax.Precision.HIGHEST

_B, _N = 2, 1024
_PLANES = [32, 64, 128, 256, 512]
_NSAMPLE = [8, 16, 16, 16, 16]
_EPS = 1e-5
_MT = [128, 64, 64, 64, 64]
_MA = [128, 128, 128, 64, 32]


def _dot(a, b, precision=None):
    return jax.lax.dot_general(a, b, (((1,), (0,)), ((), ())),
                               precision=precision, preferred_element_type=F32)


def _knn_body(xm_ref, xat_ref, idx_ref, *, mk, n, k):
    xm = xm_ref[0]
    xat = xat_ref[0]
    d = -2.0 * _dot(xm, xat, precision=HI)
    d = d + jnp.sum(xm * xm, axis=1, keepdims=True)
    d = d + jnp.sum(xat * xat, axis=0, keepdims=True)
    lane = jax.lax.broadcasted_iota(jnp.int32, (mk, n), 1)
    kcol = jax.lax.broadcasted_iota(jnp.int32, (mk, k), 1)
    idx_acc = jnp.zeros((mk, k), jnp.int32)
    for j in range(k):
        dmin = jnp.min(d, axis=1, keepdims=True)
        amin = jnp.min(jnp.where(d == dmin, lane, n), axis=1, keepdims=True)
        idx_acc = jnp.where(kcol == j, amin, idx_acc)
        d = jnp.where(lane == amin, jnp.float32(jnp.inf), d)
    idx_ref[0] = idx_acc


def _knn(xyz, xyz_t, mk=256):
    return pl.pallas_call(
        functools.partial(_knn_body, mk=mk, n=_N, k=16),
        grid=(_B, _N // mk),
        in_specs=[
            pl.BlockSpec((1, mk, 3), lambda b, i: (b, i, 0)),
            pl.BlockSpec((1, 3, _N), lambda b, i: (b, 0, 0)),
        ],
        out_specs=pl.BlockSpec((1, mk, 16), lambda b, i: (b, i, 0)),
        out_shape=jax.ShapeDtypeStruct((_B, _N, 16), jnp.int32),
    )(xyz, xyz_t)


def _td_body(ftab_ref, idx_ref, w_ref, hmax_ref, hmin_ref, stats_ref,
             *, mt, n, k, cout):
    step = pl.program_id(0) * pl.num_programs(1) + pl.program_id(1)
    r = mt * k
    ftab = ftab_ref[0]
    idxv = idx_ref[0]
    lane = jax.lax.broadcasted_iota(jnp.int32, (r, n), 1)
    onehot = (idxv == lane).astype(F32)
    g = _dot(onehot, ftab)
    h = _dot(g, w_ref[...])
    h3 = h.reshape(mt, k, cout)
    hmax_ref[0] = jnp.max(h3, axis=1)
    hmin_ref[0] = jnp.min(h3, axis=1)

    @pl.when(step == 0)
    def _():
        stats_ref[...] = jnp.zeros_like(stats_ref)

    stats_ref[0:1, :] += jnp.sum(h, axis=0, keepdims=True)
    stats_ref[1:2, :] += jnp.sum(h * h, axis=0, keepdims=True)


def _td(feature, idx_flat, w, *, k, mt):
    cin = feature.shape[-1]
    cout = w.shape[-1]
    return pl.pallas_call(
        functools.partial(_td_body, mt=mt, n=_N, k=k, cout=cout),
        grid=(_B, _N // mt),
        in_specs=[
            pl.BlockSpec((1, _N, cin), lambda b, i: (b, 0, 0)),
            pl.BlockSpec((1, mt * k, 1), lambda b, i: (b, i, 0)),
            pl.BlockSpec((cin, cout), lambda b, i: (0, 0)),
        ],
        out_specs=[
            pl.BlockSpec((1, mt, cout), lambda b, i: (b, i, 0)),
            pl.BlockSpec((1, mt, cout), lambda b, i: (b, i, 0)),
            pl.BlockSpec((8, cout), lambda b, i: (0, 0)),
        ],
        out_shape=[
            jax.ShapeDtypeStruct((_B, _N, cout), F32),
            jax.ShapeDtypeStruct((_B, _N, cout), F32),
            jax.ShapeDtypeStruct((8, cout), F32),
        ],
    )(feature, idx_flat, w)


def _proj_body(hmax_ref, hmin_ref, stats_ref, gamma_ref, beta_ref,
               wq_ref, wk_ref, wv_ref, f_ref, q_ref, kf_ref, vf_ref, *, cnt):
    s1 = stats_ref[0:1, :]
    s2 = stats_ref[1:2, :]
    m = s1 / cnt
    v = s2 / cnt - m * m
    a = gamma_ref[...] / jnp.sqrt(v + _EPS)
    c = beta_ref[...] - a * m
    hmax = hmax_ref[0]
    hmin = hmin_ref[0]
    f = jnp.maximum(jnp.maximum(a * hmax + c, a * hmin + c), 0.0)
    f_ref[0] = f
    q_ref[0] = _dot(f, wq_ref[...])
    kf_ref[0] = _dot(f, wk_ref[...])
    vf_ref[0] = _dot(f, wv_ref[...])


def _proj(hmax, hmin, stats, gamma, beta, wq, wk, wv, *, cnt, mp=128):
    d = wq.shape[0]
    spec_t = pl.BlockSpec((1, mp, d), lambda b, i: (b, i, 0))
    spec_w = pl.BlockSpec((d, d), lambda b, i: (0, 0))
    spec_v = pl.BlockSpec((1, d), lambda b, i: (0, 0))
    out_sds = jax.ShapeDtypeStruct((_B, _N, d), F32)
    return pl.pallas_call(
        functools.partial(_proj_body, cnt=float(cnt)),
        grid=(_B, _N // mp),
        in_specs=[
            spec_t, spec_t,
            pl.BlockSpec((8, d), lambda b, i: (0, 0)),
            spec_v, spec_v, spec_w, spec_w, spec_w,
        ],
        out_specs=[spec_t, spec_t, spec_t, spec_t],
        out_shape=[out_sds, out_sds, out_sds, out_sds],
    )(hmax, hmin, stats, gamma.reshape(1, d), beta.reshape(1, d), wq, wk, wv)


def _attn_body(idx_ref, q_ref, f_ref, xm_ref, kf_tab_ref, vf_tab_ref,
               xyz_tab_ref, wp1_ref, bp1_ref, wp2_ref, bp2_ref, wa1_ref,
               ba1_ref, wa2_ref, ba2_ref, wf_ref, bf_ref, out_ref,
               *, m, n, k, d):
    r = m * k
    idxv = idx_ref[0]
    lane = jax.lax.broadcasted_iota(jnp.int32, (r, n), 1)
    onehot = (idxv == lane).astype(F32)
    kf = _dot(onehot, kf_tab_ref[0])
    vf = _dot(onehot, vf_tab_ref[0])
    nxyz = _dot(onehot, xyz_tab_ref[0])
    xm = xm_ref[0]
    xm_exp = jnp.broadcast_to(xm[:, None, :], (m, k, 3)).reshape(r, 3)
    rel = xm_exp - nxyz
    ph = jnp.maximum(_dot(rel, wp1_ref[...]) + bp1_ref[...], 0.0)
    pos = _dot(ph, wp2_ref[...]) + bp2_ref[...]
    q = q_ref[0]
    q_exp = jnp.broadcast_to(q[:, None, :], (m, k, d)).reshape(r, d)
    u = q_exp - kf + pos
    hid = jnp.maximum(_dot(u, wa1_ref[...]) + ba1_ref[...], 0.0)
    attn = _dot(hid, wa2_ref[...]) + ba2_ref[...]
    a3 = (attn / jnp.sqrt(jnp.float32(d))).reshape(m, k, d)
    a3 = a3 - jax.lax.stop_gradient(jnp.max(a3, axis=1, keepdims=True))
    e = jnp.exp(a3)
    sm = e / jnp.sum(e, axis=1, keepdims=True)
    v3 = (vf + pos).reshape(m, k, d)
    nf = jnp.sum(sm * v3, axis=1)
    out_ref[0] = _dot(nf, wf_ref[...]) + bf_ref[...] + f_ref[0]


def _attn(idx_flat, q, f, xyz, kf, vf, bp, *, k, m):
    d = q.shape[-1]
    spec_t = pl.BlockSpec((1, m, d), lambda b, i: (b, i, 0))
    spec_v = lambda dim: pl.BlockSpec((1, dim), lambda b, i: (0, 0))
    spec_w = lambda din, dout: pl.BlockSpec((din, dout), lambda b, i: (0, 0))
    return pl.pallas_call(
        functools.partial(_attn_body, m=m, n=_N, k=k, d=d),
        grid=(_B, _N // m),
        in_specs=[
            pl.BlockSpec((1, m * k, 1), lambda b, i: (b, i, 0)),
            spec_t,
            spec_t,
            pl.BlockSpec((1, m, 3), lambda b, i: (b, i, 0)),
            pl.BlockSpec((1, _N, d), lambda b, i: (b, 0, 0)),
            pl.BlockSpec((1, _N, d), lambda b, i: (b, 0, 0)),
            pl.BlockSpec((1, _N, 3), lambda b, i: (b, 0, 0)),
            spec_w(3, 64), spec_v(64),
            spec_w(64, d), spec_v(d),
            spec_w(d, 4 * d), spec_v(4 * d),
            spec_w(4 * d, d), spec_v(d),
            spec_w(d, d), spec_v(d),
        ],
        out_specs=spec_t,
        out_shape=jax.ShapeDtypeStruct((_B, _N, d), F32),
    )(idx_flat, q, f, xyz, kf, vf, xyz,
      bp["Wp1"], bp["bp1"].reshape(1, 64),
      bp["Wp2"], bp["bp2"].reshape(1, d),
      bp["Wa1"], bp["ba1"].reshape(1, 4 * d),
      bp["Wa2"], bp["ba2"].reshape(1, d),
      bp["Wf"], bp["bf"].reshape(1, d))


def _head_body(f_ref, w1_ref, b1_ref, g1_ref, be1_ref, w2_ref, b2_ref,
               g2_ref, be2_ref, w3_ref, b3_ref, out_ref):
    rows = [jnp.mean(f_ref[b], axis=0, keepdims=True) for b in range(_B)]
    h = jnp.concatenate(rows, axis=0)

    def bn(x, g, be):
        mu = jnp.mean(x, axis=0, keepdims=True)
        va = jnp.mean((x - mu) * (x - mu), axis=0, keepdims=True)
        return g * (x - mu) / jnp.sqrt(va + _EPS) + be

    h = jnp.maximum(bn(_dot(h, w1_ref[...]) + b1_ref[...],
                       g1_ref[...], be1_ref[...]), 0.0)
    h = jnp.maximum(bn(_dot(h, w2_ref[...]) + b2_ref[...],
                       g2_ref[...], be2_ref[...]), 0.0)
    out_ref[...] = _dot(h, w3_ref[...]) + b3_ref[...]


def _head(feature, c):
    spec_w = lambda din, dout: pl.BlockSpec((din, dout), lambda: (0, 0))
    spec_v = lambda dim: pl.BlockSpec((1, dim), lambda: (0, 0))
    return pl.pallas_call(
        _head_body,
        in_specs=[
            pl.BlockSpec((_B, _N, 512), lambda: (0, 0, 0)),
            spec_w(512, 256), spec_v(256), spec_v(256), spec_v(256),
            spec_w(256, 128), spec_v(128), spec_v(128), spec_v(128),
            spec_w(128, 40), spec_v(40),
        ],
        out_specs=pl.BlockSpec((_B, 40), lambda: (0, 0)),
        out_shape=jax.ShapeDtypeStruct((_B, 40), F32),
    )(feature,
      c["W1"], c["b1"].reshape(1, 256), c["g1"].reshape(1, 256),
      c["be1"].reshape(1, 256),
      c["W2"], c["b2"].reshape(1, 128), c["g2"].reshape(1, 128),
      c["be2"].reshape(1, 128),
      c["W3"], c["b3"].reshape(1, 40))


def kernel(x, params):
    xyz = x[..., :3]
    xyz_t = jnp.swapaxes(xyz, 1, 2)
    idx16 = _knn(xyz, xyz_t)
    idx_flat = {
        16: idx16.reshape(_B, _N * 16, 1),
        8: idx16[:, :, :8].reshape(_B, _N * 8, 1),
    }
    feature = x
    for i, st in enumerate(params["stages"]):
        k = _NSAMPLE[i]
        td = st["td"]
        hmax, hmin, stats = _td(feature, idx_flat[k], td["W"], k=k, mt=_MT[i])
        bp = st["blocks"][0]
        f, q, kf, vf = _proj(hmax, hmin, stats, td["gamma"], td["beta"],
                             bp["Wq"], bp["Wk"], bp["Wv"], cnt=_B * _N * k)
        feature = _attn(idx_flat[k], q, f, xyz, kf, vf, bp, k=k, m=_MA[i])
    return _head(feature, params["cls"])
